# Initial kernel scaffold; baseline (speedup 1.0000x reference)
import jax
import jax.numpy as jnp
from jax.experimental import pallas as pl


def kernel(input, fspace):
    raise NotImplementedError("write your pallas kernel here")


# speedup vs baseline: 5.6459x; 5.6459x over previous
import functools

import jax
import jax.numpy as jnp
from jax import lax
from jax.experimental import pallas as pl
from jax.experimental.pallas import tpu as pltpu
from jax.experimental.pallas import tpu_sc as plsc

ROWS = 4096
COLS = 4096
ROW_BLOCK = 256
NUM_BLOCKS = ROWS // ROW_BLOCK

NUM_CORES = 2
NUM_SUBCORES = 16
NUM_WORKERS = NUM_CORES * NUM_SUBCORES
CHUNK = ROWS // NUM_WORKERS


def _argmax_body(x_hbm, idx_ref, buf, sem):
    i = pl.program_id(0)
    slot = lax.rem(i, 2)
    nxt = lax.rem(i + 1, 2)

    @pl.when(i == 0)
    def _():
        pltpu.make_async_copy(
            x_hbm.at[pl.ds(0, ROW_BLOCK), 0], buf.at[0], sem.at[0]
        ).start()

    @pl.when(i + 1 < NUM_BLOCKS)
    def _():
        pltpu.make_async_copy(
            x_hbm.at[pl.ds((i + 1) * ROW_BLOCK, ROW_BLOCK), 0],
            buf.at[nxt],
            sem.at[nxt],
        ).start()

    pltpu.make_async_copy(
        x_hbm.at[pl.ds(i * ROW_BLOCK, ROW_BLOCK), 0], buf.at[slot], sem.at[slot]
    ).wait()

    x = buf[slot]
    m = jnp.max(x, axis=1, keepdims=True)
    col = lax.broadcasted_iota(jnp.int32, x.shape, 1)
    cand = jnp.where(x == m, col, COLS)
    idx_ref[0, 0, :] = jnp.min(cand, axis=1)


_argmax_call = pl.pallas_call(
    _argmax_body,
    grid=(NUM_BLOCKS,),
    in_specs=[pl.BlockSpec(memory_space=pltpu.MemorySpace.HBM)],
    out_specs=pl.BlockSpec((1, 1, ROW_BLOCK), lambda i: (i, 0, 0)),
    out_shape=jax.ShapeDtypeStruct((NUM_BLOCKS, 1, ROW_BLOCK), jnp.int32),
    scratch_shapes=[
        pltpu.VMEM((2, ROW_BLOCK, COLS), jnp.float32),
        pltpu.SemaphoreType.DMA((2,)),
    ],
)


@functools.partial(
    pl.kernel,
    out_type=jax.ShapeDtypeStruct((ROWS,), jnp.float32),
    scratch_types=[
        pltpu.VMEM((CHUNK,), jnp.int32),
        pltpu.VMEM((CHUNK,), jnp.float32),
        pltpu.SemaphoreType.DMA,
    ],
    mesh=plsc.VectorSubcoreMesh(
        core_axis_name="c", subcore_axis_name="s", num_cores=NUM_CORES
    ),
)
def _sc_gather(idx_hbm, fsp_hbm, out_hbm, idx_v, out_v, sem):
    wid = lax.axis_index("s") * NUM_CORES + lax.axis_index("c")
    base = wid * CHUNK
    pltpu.sync_copy(idx_hbm.at[pl.ds(base, CHUNK)], idx_v)
    pltpu.async_copy(fsp_hbm.at[idx_v], out_v, sem).wait()
    pltpu.sync_copy(out_v, out_hbm.at[pl.ds(base, CHUNK)])


def kernel(input, fspace):
    idx = _argmax_call(input).reshape(ROWS)
    return _sc_gather(idx, fspace)


# speedup vs baseline: 6.5906x; 1.1673x over previous
import functools

import jax
import jax.numpy as jnp
from jax import lax
from jax.experimental import pallas as pl
from jax.experimental.pallas import tpu as pltpu
from jax.experimental.pallas import tpu_sc as plsc

ROWS = 4096
COLS = 4096
ROW_BLOCK = 128
NUM_BLOCKS = ROWS // ROW_BLOCK
NBUF = 4

NUM_CORES = 2
NUM_SUBCORES = 16
NUM_WORKERS = NUM_CORES * NUM_SUBCORES
CHUNK = ROWS // NUM_WORKERS


def _argmax_body(x_hbm, idx_ref, buf, sem):
    i = pl.program_id(0)
    slot = lax.rem(i, NBUF)

    @pl.when(i == 0)
    def _():
        for j in range(NBUF - 1):
            pltpu.make_async_copy(
                x_hbm.at[pl.ds(j * ROW_BLOCK, ROW_BLOCK), 0],
                buf.at[j],
                sem.at[j],
            ).start()

    @pl.when(i + NBUF - 1 < NUM_BLOCKS)
    def _():
        nxt = lax.rem(i + NBUF - 1, NBUF)
        pltpu.make_async_copy(
            x_hbm.at[pl.ds((i + NBUF - 1) * ROW_BLOCK, ROW_BLOCK), 0],
            buf.at[nxt],
            sem.at[nxt],
        ).start()

    pltpu.make_async_copy(
        x_hbm.at[pl.ds(i * ROW_BLOCK, ROW_BLOCK), 0], buf.at[slot], sem.at[slot]
    ).wait()

    x = buf[slot]
    m = jnp.max(x, axis=1, keepdims=True)
    col = lax.broadcasted_iota(jnp.int32, x.shape, 1)
    cand = jnp.where(x == m, col, COLS)
    idx_ref[0, 0, :] = jnp.min(cand, axis=1)


_argmax_call = pl.pallas_call(
    _argmax_body,
    grid=(NUM_BLOCKS,),
    in_specs=[pl.BlockSpec(memory_space=pltpu.MemorySpace.HBM)],
    out_specs=pl.BlockSpec((1, 1, ROW_BLOCK), lambda i: (i, 0, 0)),
    out_shape=jax.ShapeDtypeStruct((NUM_BLOCKS, 1, ROW_BLOCK), jnp.int32),
    scratch_shapes=[
        pltpu.VMEM((NBUF, ROW_BLOCK, COLS), jnp.float32),
        pltpu.SemaphoreType.DMA((NBUF,)),
    ],
)


@functools.partial(
    pl.kernel,
    out_type=jax.ShapeDtypeStruct((ROWS,), jnp.float32),
    scratch_types=[
        pltpu.VMEM((CHUNK,), jnp.int32),
        pltpu.VMEM((CHUNK,), jnp.float32),
        pltpu.SemaphoreType.DMA,
    ],
    mesh=plsc.VectorSubcoreMesh(
        core_axis_name="c", subcore_axis_name="s", num_cores=NUM_CORES
    ),
)
def _sc_gather(idx_hbm, fsp_hbm, out_hbm, idx_v, out_v, sem):
    wid = lax.axis_index("s") * NUM_CORES + lax.axis_index("c")
    base = wid * CHUNK
    pltpu.sync_copy(idx_hbm.at[pl.ds(base, CHUNK)], idx_v)
    pltpu.async_copy(fsp_hbm.at[idx_v], out_v, sem).wait()
    pltpu.sync_copy(out_v, out_hbm.at[pl.ds(base, CHUNK)])


def kernel(input, fspace):
    idx = _argmax_call(input).reshape(ROWS)
    return _sc_gather(idx, fspace)
